# baseline (speedup 1.0000x reference)
import functools
import math

import jax
import jax.numpy as jnp
import numpy as np
from jax import lax
from jax.experimental import pallas as pl
from jax.experimental.pallas import tpu as pltpu
from jax.experimental.pallas import tpu_sc as plsc

_VOCAB = 100000
_D = 1024
_B = 4
_S = 2048
_SCALE = math.sqrt(_D)

_NW = 32
_P_PER_W = _S // _NW
_PC = 16
_NCH = _P_PER_W // _PC
_LANES = 16


def _sin_pe(max_len, d_model):
    pos = np.arange(max_len, dtype=np.float32)[:, None]
    div = np.exp(
        np.arange(0, d_model, 2, dtype=np.float32) * (-math.log(10000.0) / d_model)
    )
    pe = np.zeros((max_len, d_model), dtype=np.float32)
    pe[:, 0::2] = np.sin(pos * div)
    pe[:, 1::2] = np.cos(pos * div)
    return pe


_PE = _sin_pe(_S, _D)

_mesh = plsc.VectorSubcoreMesh(core_axis_name="c", subcore_axis_name="s")


@functools.partial(
    pl.kernel,
    mesh=_mesh,
    out_type=jax.ShapeDtypeStruct((_B * _S, _D), jnp.float32),
    scratch_types=[
        pltpu.VMEM((_PC,), jnp.int32),
        pltpu.VMEM((_PC, _D), jnp.float32),
        pltpu.VMEM((_PC, _D), jnp.float32),
        pltpu.VMEM((_PC, _D), jnp.float32),
        pltpu.VMEM((_PC, _D), jnp.float32),
        pltpu.VMEM((_PC, _D), jnp.float32),
        pltpu.SemaphoreType.DMA,
    ],
)
def _emb_kernel(ids_hbm, w_hbm, pe_hbm, out_hbm,
                idx_v, pe_v, r0, r1, r2, r3, sem):
    rows = (r0, r1, r2, r3)
    wid = lax.axis_index("s") * 2 + lax.axis_index("c")
    base_p = wid * _P_PER_W

    for c in range(_NCH):
        pos0 = base_p + c * _PC
        pltpu.sync_copy(pe_hbm.at[pl.ds(pos0, _PC)], pe_v)
        for b in range(_B):
            pltpu.sync_copy(ids_hbm.at[pl.ds(b * _S + pos0, _PC)], idx_v)
            pltpu.async_copy(w_hbm.at[idx_v], rows[b], sem).wait()

        def body_r(r, _):
            def body_j(j, _):
                sl = pl.ds(j * _LANES, _LANES)
                pv = pe_v[r, sl]
                for rb in rows:
                    rb[r, sl] = rb[r, sl] * _SCALE + pv
                return 0

            return lax.fori_loop(0, _D // _LANES, body_j, 0)

        lax.fori_loop(0, _PC, body_r, 0)

        for b in range(_B):
            pltpu.sync_copy(rows[b], out_hbm.at[pl.ds(b * _S + pos0, _PC)])


def kernel(token_ids, W):
    ids = token_ids.reshape(-1).astype(jnp.int32)
    pe = jnp.asarray(_PE)
    out = _emb_kernel(ids, W, pe)
    return out.reshape(_B, _S, _D)


# speedup vs baseline: 1.5987x; 1.5987x over previous
import functools
import math

import jax
import jax.numpy as jnp
import numpy as np
from jax import lax
from jax.experimental import pallas as pl
from jax.experimental.pallas import tpu as pltpu
from jax.experimental.pallas import tpu_sc as plsc

_VOCAB = 100000
_D = 1024
_B = 4
_S = 2048
_SCALE = math.sqrt(_D)

_NW = 32
_P_PER_W = _S // _NW
_PC = 16
_NCH = _P_PER_W // _PC
_NITEM = _NCH * _B
_LANES = 16


def _sin_pe(max_len, d_model):
    pos = np.arange(max_len, dtype=np.float32)[:, None]
    div = np.exp(
        np.arange(0, d_model, 2, dtype=np.float32) * (-math.log(10000.0) / d_model)
    )
    pe = np.zeros((max_len, d_model), dtype=np.float32)
    pe[:, 0::2] = np.sin(pos * div)
    pe[:, 1::2] = np.cos(pos * div)
    return pe


_PE = _sin_pe(_S, _D)

_mesh = plsc.VectorSubcoreMesh(core_axis_name="c", subcore_axis_name="s")


@functools.partial(
    pl.kernel,
    mesh=_mesh,
    out_type=jax.ShapeDtypeStruct((_B * _S, _D), jnp.float32),
    scratch_types=[
        pltpu.VMEM((_B, _P_PER_W), jnp.int32),
        pltpu.VMEM((_PC, _D), jnp.float32),
        pltpu.VMEM((_PC, _D), jnp.float32),
        pltpu.VMEM((_PC, _D), jnp.float32),
        pltpu.VMEM((_PC, _D), jnp.float32),
        pltpu.VMEM((_PC, _D), jnp.float32),
        pltpu.VMEM((_PC, _D), jnp.float32),
        pltpu.SemaphoreType.DMA,
        pltpu.SemaphoreType.DMA,
        pltpu.SemaphoreType.DMA,
        pltpu.SemaphoreType.DMA,
        pltpu.SemaphoreType.DMA,
        pltpu.SemaphoreType.DMA,
        pltpu.SemaphoreType.DMA,
        pltpu.SemaphoreType.DMA,
        pltpu.SemaphoreType.DMA,
        pltpu.SemaphoreType.DMA,
    ],
)
def _emb_kernel(ids_hbm, w_hbm, pe_hbm, out_hbm,
                idx_v, r0, r1, r2, r3, pe0, pe1,
                g0, g1, g2, g3, s0, s1, s2, s3, psem0, psem1):
    rows = (r0, r1, r2, r3)
    pes = (pe0, pe1)
    gsems = (g0, g1, g2, g3)
    ssems = (s0, s1, s2, s3)
    psems = (psem0, psem1)

    wid = lax.axis_index("s") * 2 + lax.axis_index("c")
    base_p = wid * _P_PER_W

    def gather_copy(i):
        b, c = i % _B, i // _B
        return pltpu.make_async_copy(
            w_hbm.at[idx_v.at[b, pl.ds(c * _PC, _PC)]], rows[b], gsems[b])

    def store_copy(i):
        b, c = i % _B, i // _B
        return pltpu.make_async_copy(
            rows[b], out_hbm.at[pl.ds(b * _S + base_p + c * _PC, _PC)], ssems[b])

    def pe_copy(c):
        return pltpu.make_async_copy(
            pe_hbm.at[pl.ds(base_p + c * _PC, _PC)], pes[c % 2], psems[c % 2])

    for b in range(_B):
        pltpu.sync_copy(ids_hbm.at[pl.ds(b * _S + base_p, _P_PER_W)],
                        idx_v.at[b])
    pe_copy(0).start()
    gather_copy(0).start()
    gather_copy(1).start()

    for i in range(_NITEM):
        b, c = i % _B, i // _B
        if i + 2 < _NITEM:
            if i >= 2:
                store_copy(i - 2).wait()
            gather_copy(i + 2).start()
        if b == 0 and c + 1 < _NCH:
            pe_copy(c + 1).start()
        if b == 0:
            pe_copy(c).wait()
        gather_copy(i).wait()

        rb, pb = rows[b], pes[c % 2]

        def body_r(r, _):
            def body_j(j, _):
                for jj in range(4):
                    sl = pl.ds((j * 4 + jj) * _LANES, _LANES)
                    rb[r, sl] = rb[r, sl] * _SCALE + pb[r, sl]
                return 0

            return lax.fori_loop(0, _D // (_LANES * 4), body_j, 0)

        lax.fori_loop(0, _PC, body_r, 0)
        store_copy(i).start()

    for i in range(_NITEM - 4, _NITEM):
        store_copy(i).wait()


def kernel(token_ids, W):
    ids = token_ids.reshape(-1).astype(jnp.int32)
    pe = jnp.asarray(_PE)
    out = _emb_kernel(ids, W, pe)
    return out.reshape(_B, _S, _D)
